# baseline (speedup 1.0000x reference)
import functools

import jax
import jax.numpy as jnp
from jax import lax
from jax.experimental import pallas as pl
from jax.experimental.pallas import tpu as pltpu
from jax.experimental.pallas import tpu_sc as plsc

NC = 2
NS = 16
L = 16
NW = NC * NS

BATCH = 16384
EMBED_K = 16
BPW = BATCH // NW
C = 4
SLOTS = 7
NCHUNK = BPW // C


def _sc_body(ui_hbm, ii_hbm, wt_hbm, ht_hbm, out_hbm,
             uiv, iiv, ubuf, vbuf, outv, semu, semv):
    wid = lax.axis_index("s") * NC + lax.axis_index("c")
    base = wid * BPW

    pltpu.sync_copy(ui_hbm.at[pl.ds(base, BPW)], uiv.at[pl.ds(0, BPW)])
    pltpu.sync_copy(ii_hbm.at[pl.ds(base, BPW)], iiv.at[pl.ds(0, BPW)])

    lanes = lax.iota(jnp.int32, L)

    def chunk_vecs(c):
        return uiv[pl.ds(c * C, L)], iiv[pl.ds(c * C, L)]

    def fire(c, slot):
        uvec, ivec = chunk_vecs(c)
        for k in range(C):
            r = uvec[k]
            rb = pl.multiple_of(r & ~127, 128)
            pltpu.async_copy(
                wt_hbm.at[:, pl.ds(rb, 128)], ubuf.at[slot, k], semu.at[slot])
            q = ivec[k]
            qb = pl.multiple_of(q & ~127, 128)
            pltpu.async_copy(
                ht_hbm.at[:, pl.ds(qb, 128)], vbuf.at[slot, k], semv.at[slot])

    def drain(slot):
        for k in range(C):
            pltpu.make_async_copy(
                wt_hbm.at[:, pl.ds(0, 128)], ubuf.at[slot, k],
                semu.at[slot]).wait()
            pltpu.make_async_copy(
                ht_hbm.at[:, pl.ds(0, 128)], vbuf.at[slot, k],
                semv.at[slot]).wait()

    def compute(c, slot, off):
        uvec, ivec = chunk_vecs(c)
        dots = jnp.zeros((L,), jnp.float32)
        for k in range(C):
            rm = uvec[k] & 127
            qm = ivec[k] & 127
            u = plsc.load_gather(
                ubuf.at[slot, k], [lanes, jnp.zeros((L,), jnp.int32) + rm])
            v = plsc.load_gather(
                vbuf.at[slot, k], [lanes, jnp.zeros((L,), jnp.int32) + qm])
            s = jnp.sum(u * v)
            dots = jnp.where(lanes == off + k, s, dots)
        return dots

    AHEAD = SLOTS - 1
    CPB = L // C
    for j in range(AHEAD):
        fire(j, j)

    def chunk_body(c, carry):
        slot = lax.rem(c, SLOTS)

        @pl.when(c + AHEAD < NCHUNK)
        def _():
            fire(c + AHEAD, lax.rem(c + AHEAD, SLOTS))

        drain(slot)
        blk = (c // CPB) * L
        h = lax.rem(c, CPB) * C
        dots = compute(c, slot, h)

        ob = outv[pl.ds(blk, L)]
        ob = jnp.where((lanes >= h) & (lanes < h + C), dots, ob)
        outv[pl.ds(blk, L)] = ob
        return carry

    lax.fori_loop(0, NCHUNK, chunk_body, jnp.int32(0))

    pltpu.sync_copy(outv, out_hbm.at[pl.ds(base, BPW)])


@jax.jit
def _mf_dot(x, W, H):
    ui = x[:, 0]
    ii = x[:, 1]
    wt = W.T
    ht = H.T
    mesh = plsc.VectorSubcoreMesh(core_axis_name="c", subcore_axis_name="s")
    return pl.kernel(
        _sc_body,
        out_type=jax.ShapeDtypeStruct((BATCH,), jnp.float32),
        mesh=mesh,
        compiler_params=pltpu.CompilerParams(
            needs_layout_passes=False, use_tc_tiling_on_sc=True),
        scratch_types=[
            pltpu.VMEM((BPW + L,), jnp.int32),
            pltpu.VMEM((BPW + L,), jnp.int32),
            pltpu.VMEM((SLOTS, C, EMBED_K, 128), jnp.float32),
            pltpu.VMEM((SLOTS, C, EMBED_K, 128), jnp.float32),
            pltpu.VMEM((BPW,), jnp.float32),
            pltpu.SemaphoreType.DMA((SLOTS,)),
            pltpu.SemaphoreType.DMA((SLOTS,)),
        ],
    )(ui, ii, wt, ht)


def kernel(x, W, H):
    return _mf_dot(x.astype(jnp.int32), W, H)


# speedup vs baseline: 1.0049x; 1.0049x over previous
import jax
import jax.numpy as jnp
from jax import lax
from jax.experimental import pallas as pl
from jax.experimental.pallas import tpu as pltpu
from jax.experimental.pallas import tpu_sc as plsc

NC = 2
NS = 16
L = 16
NW = NC * NS

BATCH = 16384
EMBED_K = 16
BPW = BATCH // NW
C = 4
SLOTS = 6
NCHUNK = BPW // C


def _sc_body(ui_hbm, ii_hbm, wt_hbm, ht_hbm, out_hbm,
             uiv, iiv, ubuf, vbuf, outv, semu, semv):
    wid = lax.axis_index("s") * NC + lax.axis_index("c")
    base = wid * BPW

    pltpu.sync_copy(ui_hbm.at[pl.ds(base, BPW)], uiv.at[pl.ds(0, BPW)])
    pltpu.sync_copy(ii_hbm.at[pl.ds(base, BPW)], iiv.at[pl.ds(0, BPW)])

    lanes = lax.iota(jnp.int32, L)

    def chunk_vecs(c):
        return uiv[pl.ds(c * C, L)], iiv[pl.ds(c * C, L)]

    def fire(c, slot):
        uvec, ivec = chunk_vecs(c)
        for k in range(C):
            r = uvec[k]
            rb = pl.multiple_of(r & ~127, 128)
            pltpu.async_copy(
                wt_hbm.at[:, pl.ds(rb, 128)], ubuf.at[slot, k], semu.at[slot])
            q = ivec[k]
            qb = pl.multiple_of(q & ~127, 128)
            pltpu.async_copy(
                ht_hbm.at[:, pl.ds(qb, 128)], vbuf.at[slot, k], semv.at[slot])

    def drain(slot):
        for k in range(C):
            pltpu.make_async_copy(
                wt_hbm.at[:, pl.ds(0, 128)], ubuf.at[slot, k],
                semu.at[slot]).wait()
            pltpu.make_async_copy(
                ht_hbm.at[:, pl.ds(0, 128)], vbuf.at[slot, k],
                semv.at[slot]).wait()

    def compute(c, slot, off):
        uvec, ivec = chunk_vecs(c)
        dots = jnp.zeros((L,), jnp.float32)
        for k in range(C):
            rm = uvec[k] & 127
            qm = ivec[k] & 127
            u = plsc.load_gather(
                ubuf.at[slot, k], [lanes, jnp.zeros((L,), jnp.int32) + rm])
            v = plsc.load_gather(
                vbuf.at[slot, k], [lanes, jnp.zeros((L,), jnp.int32) + qm])
            s = jnp.sum(u * v)
            dots = jnp.where(lanes == off + k, s, dots)
        return dots

    AHEAD = SLOTS - 1
    CPB = L // C
    for j in range(AHEAD):
        fire(j, j)

    def chunk_body(c, carry):
        slot = lax.rem(c, SLOTS)

        @pl.when(c + AHEAD < NCHUNK)
        def _():
            fire(c + AHEAD, lax.rem(c + AHEAD, SLOTS))

        drain(slot)
        blk = (c // CPB) * L
        h = lax.rem(c, CPB) * C
        dots = compute(c, slot, h)

        ob = outv[pl.ds(blk, L)]
        ob = jnp.where((lanes >= h) & (lanes < h + C), dots, ob)
        outv[pl.ds(blk, L)] = ob
        return carry

    lax.fori_loop(0, NCHUNK, chunk_body, jnp.int32(0))

    pltpu.sync_copy(outv, out_hbm.at[pl.ds(base, BPW)])


@jax.jit
def _mf_dot(x, W, H):
    ui = x[:, 0]
    ii = x[:, 1]
    wt = W.T
    ht = H.T
    mesh = plsc.VectorSubcoreMesh(core_axis_name="c", subcore_axis_name="s")
    return pl.kernel(
        _sc_body,
        out_type=jax.ShapeDtypeStruct((BATCH,), jnp.float32),
        mesh=mesh,
        compiler_params=pltpu.CompilerParams(
            needs_layout_passes=False, use_tc_tiling_on_sc=True),
        scratch_types=[
            pltpu.VMEM((BPW + L,), jnp.int32),
            pltpu.VMEM((BPW + L,), jnp.int32),
            pltpu.VMEM((SLOTS, C, EMBED_K, 128), jnp.float32),
            pltpu.VMEM((SLOTS, C, EMBED_K, 128), jnp.float32),
            pltpu.VMEM((BPW,), jnp.float32),
            pltpu.SemaphoreType.DMA((SLOTS,)),
            pltpu.SemaphoreType.DMA((SLOTS,)),
        ],
    )(ui, ii, wt, ht)


def kernel(x, W, H):
    return _mf_dot(x.astype(jnp.int32), W, H)
